# baseline (speedup 1.0000x reference)
import functools

import jax
import jax.numpy as jnp
from jax import lax
from jax.experimental import pallas as pl
from jax.experimental.pallas import tpu as pltpu
from jax.experimental.pallas import tpu_sc as plsc


def _sc_gather_body(sidx_hbm, stab_hbm, cidx_hbm, ctab_hbm,
                    kp_hbm, sout_hbm, idx_v, rows_v, sem, idx_v2, rows_v2,
                    sem2, *, n_real):
    wid = lax.axis_index("s") * 2 + lax.axis_index("c")
    nb = idx_v.shape[0]
    T, B = cidx_hbm.shape[1], cidx_hbm.shape[2]
    nb_s = sidx_hbm.shape[0] * sidx_hbm.shape[1] // 32
    i = wid // 8
    base_s = (wid % 8) * nb_s
    pltpu.sync_copy(sidx_hbm.at[i, pl.ds(base_s, nb_s)], idx_v.at[pl.ds(0, nb_s)])
    pltpu.async_copy(stab_hbm.at[idx_v.at[pl.ds(0, nb_s)]],
                     rows_v.at[pl.ds(0, nb_s)], sem).wait()
    pltpu.sync_copy(rows_v.at[pl.ds(0, nb_s)],
                    sout_hbm.at[pl.ds(base_s, nb_s), i])

    bchunks = B // nb
    per_w = T * 2 * bchunks // 32

    def coords(c):
        return c // (2 * bchunks), (c // bchunks) % 2, (c % bchunks) * nb

    def gather_one(c):
        t, j, base = coords(c)
        pltpu.sync_copy(cidx_hbm.at[j, t, pl.ds(base, nb)], idx_v)
        pltpu.async_copy(ctab_hbm.at[idx_v], rows_v, sem).wait()
        pltpu.sync_copy(rows_v, kp_hbm.at[t, n_real + j, pl.ds(base, nb)])

    def body(p, _):
        cA = wid + 32 * (2 * p)
        cB = wid + 32 * (2 * p + 1)
        tA, jA, baseA = coords(cA)
        tB, jB, baseB = coords(cB)
        pltpu.sync_copy(cidx_hbm.at[jA, tA, pl.ds(baseA, nb)], idx_v)
        pltpu.sync_copy(cidx_hbm.at[jB, tB, pl.ds(baseB, nb)], idx_v2)
        hA = pltpu.async_copy(ctab_hbm.at[idx_v], rows_v, sem)
        hB = pltpu.async_copy(ctab_hbm.at[idx_v2], rows_v2, sem2)
        hA.wait()
        pltpu.sync_copy(rows_v, kp_hbm.at[tA, n_real + jA, pl.ds(baseA, nb)])
        hB.wait()
        pltpu.sync_copy(rows_v2, kp_hbm.at[tB, n_real + jB, pl.ds(baseB, nb)])
        return 0

    lax.fori_loop(0, per_w // 2, body, 0)
    if per_w % 2:
        gather_one(wid + 32 * (per_w - 1))


def _sc_gather(sidx_b, stab2, cidx, ctab2, n_real, n_known):
    _, T, B = cidx.shape
    n_static = sidx_b.shape[0]
    L = stab2.shape[-1]
    mesh = plsc.VectorSubcoreMesh(core_axis_name="c", subcore_axis_name="s")
    f = pl.kernel(
        functools.partial(_sc_gather_body, n_real=n_real),
        mesh=mesh,
        out_type=[
            jax.ShapeDtypeStruct((T, n_known, B, L), jnp.float32),
            jax.ShapeDtypeStruct((B, n_static, L), jnp.float32),
        ],
        scratch_types=[
            pltpu.VMEM((128,), jnp.int32),
            pltpu.VMEM((128, L), jnp.float32),
            pltpu.SemaphoreType.DMA,
            pltpu.VMEM((128,), jnp.int32),
            pltpu.VMEM((128, L), jnp.float32),
            pltpu.SemaphoreType.DMA,
        ],
    )
    return f(sidx_b, stab2, cidx, ctab2)


def _known_dense_body(x_ref, w_ref, b_ref, kp_any, out_ref):
    del kp_any
    out_ref[...] = x_ref[...] * w_ref[...] + b_ref[...]


def _obs_body(xo_ref, wo_ref, bo_ref, obs_ref):
    obs_ref[...] = (xo_ref[...] * wo_ref[...][None, None, :, :]
                    + bo_ref[...][None, None, :, :])


def kernel(static, known_real, known_categorical, observed, static_tables,
           known_cat_tables, real_W, real_b, obs_W, obs_b):
    B, T, n_real = known_real.shape
    n_obs = observed.shape[-1]
    n_cat = known_categorical.shape[-1]
    n_static = static_tables.shape[0]
    vocab, L = static_tables.shape[1], static_tables.shape[2]
    n_known = n_real + n_cat

    sidx_b = (static[:, 0, :].astype(jnp.int32).T
              + jnp.arange(n_static, dtype=jnp.int32)[:, None] * vocab)
    stab2 = static_tables.reshape(n_static * vocab, L)
    cidx = (known_categorical.transpose(2, 1, 0).astype(jnp.int32)
            + jnp.arange(n_cat, dtype=jnp.int32)[:, None, None] * vocab)
    ctab2 = known_cat_tables.reshape(n_cat * vocab, L)
    catp, static_embs = _sc_gather(sidx_b, stab2, cidx, ctab2,
                                   n_real, n_known)

    xk_t = known_real.transpose(1, 2, 0)[..., None]
    xo4 = observed[..., None]
    w4 = real_W[None, :, None, :]
    b4 = real_b[None, :, None, :]

    full = lambda shape: pl.BlockSpec(shape, lambda *a: (0,) * len(shape))
    OB = 32
    obs_p = pl.pallas_call(
        _obs_body,
        grid=(B // OB,),
        in_specs=[
            pl.BlockSpec((OB, T, n_obs, 1), lambda r: (r, 0, 0, 0)),
            full((n_obs, L)),
            full((n_obs, L)),
        ],
        out_specs=pl.BlockSpec((OB, T, n_obs, L), lambda r: (r, 0, 0, 0)),
        out_shape=jax.ShapeDtypeStruct((B, T, n_obs, L), jnp.float32),
    )(xo4, obs_W, obs_b)

    BB = 32
    known_p = pl.pallas_call(
        _known_dense_body,
        grid=(B // BB,),
        in_specs=[
            pl.BlockSpec((T, n_real, BB, 1), lambda r: (0, 0, r, 0)),
            full((1, n_real, 1, L)),
            full((1, n_real, 1, L)),
            pl.BlockSpec(memory_space=pl.ANY),
        ],
        out_specs=pl.BlockSpec((T, n_real, BB, L), lambda r: (0, 0, r, 0)),
        out_shape=jax.ShapeDtypeStruct((T, n_known, B, L), jnp.float32),
        input_output_aliases={3: 0},
    )(xk_t, w4, b4, catp)

    return (static_embs,
            jnp.transpose(known_p, (2, 0, 3, 1)),
            jnp.swapaxes(obs_p, 2, 3))


# speedup vs baseline: 1.0075x; 1.0075x over previous
import functools

import jax
import jax.numpy as jnp
from jax import lax
from jax.experimental import pallas as pl
from jax.experimental.pallas import tpu as pltpu
from jax.experimental.pallas import tpu_sc as plsc


def _sc_gather_body(sidx_hbm, stab_hbm, cidx_hbm, ctab_hbm,
                    kp_hbm, sout_hbm, idx_v, rows_v, sem, *, n_real):
    wid = lax.axis_index("s") * 2 + lax.axis_index("c")
    nb = idx_v.shape[0]
    T, B = cidx_hbm.shape[1], cidx_hbm.shape[2]
    nb_s = sidx_hbm.shape[0] * sidx_hbm.shape[1] // 32
    i = wid // 8
    base_s = (wid % 8) * nb_s
    pltpu.sync_copy(sidx_hbm.at[i, pl.ds(base_s, nb_s)], idx_v.at[pl.ds(0, nb_s)])
    pltpu.async_copy(stab_hbm.at[idx_v.at[pl.ds(0, nb_s)]],
                     rows_v.at[pl.ds(0, nb_s)], sem).wait()
    pltpu.sync_copy(rows_v.at[pl.ds(0, nb_s)],
                    sout_hbm.at[pl.ds(base_s, nb_s), i])

    bchunks = B // nb
    per_w = T * 2 * bchunks // 32

    def body(it, _):
        c = wid + 32 * it
        t = c // (2 * bchunks)
        j = (c // bchunks) % 2
        base = (c % bchunks) * nb
        pltpu.sync_copy(cidx_hbm.at[j, t, pl.ds(base, nb)], idx_v)
        pltpu.async_copy(ctab_hbm.at[idx_v], rows_v, sem).wait()
        pltpu.sync_copy(rows_v, kp_hbm.at[t, n_real + j, pl.ds(base, nb)])
        return 0

    lax.fori_loop(0, per_w, body, 0)


def _sc_gather(sidx_b, stab2, cidx, ctab2, n_real, n_known):
    _, T, B = cidx.shape
    n_static = sidx_b.shape[0]
    L = stab2.shape[-1]
    mesh = plsc.VectorSubcoreMesh(core_axis_name="c", subcore_axis_name="s")
    f = pl.kernel(
        functools.partial(_sc_gather_body, n_real=n_real),
        mesh=mesh,
        out_type=[
            jax.ShapeDtypeStruct((T, n_known, B, L), jnp.float32),
            jax.ShapeDtypeStruct((B, n_static, L), jnp.float32),
        ],
        scratch_types=[
            pltpu.VMEM((128,), jnp.int32),
            pltpu.VMEM((128, L), jnp.float32),
            pltpu.SemaphoreType.DMA,
        ],
    )
    return f(sidx_b, stab2, cidx, ctab2)


def _known_dense_body(x_ref, w_ref, b_ref, kp_any, out_ref):
    del kp_any
    out_ref[...] = x_ref[...] * w_ref[...] + b_ref[...]


def _obs_body(xo_ref, wo_ref, bo_ref, obs_ref):
    obs_ref[...] = (xo_ref[...] * wo_ref[...][None, None, :, :]
                    + bo_ref[...][None, None, :, :])


def kernel(static, known_real, known_categorical, observed, static_tables,
           known_cat_tables, real_W, real_b, obs_W, obs_b):
    B, T, n_real = known_real.shape
    n_obs = observed.shape[-1]
    n_cat = known_categorical.shape[-1]
    n_static = static_tables.shape[0]
    vocab, L = static_tables.shape[1], static_tables.shape[2]
    n_known = n_real + n_cat

    sidx_b = (static[:, 0, :].astype(jnp.int32).T
              + jnp.arange(n_static, dtype=jnp.int32)[:, None] * vocab)
    stab2 = static_tables.reshape(n_static * vocab, L)
    cidx = (known_categorical.transpose(2, 1, 0).astype(jnp.int32)
            + jnp.arange(n_cat, dtype=jnp.int32)[:, None, None] * vocab)
    ctab2 = known_cat_tables.reshape(n_cat * vocab, L)
    catp, static_embs = _sc_gather(sidx_b, stab2, cidx, ctab2,
                                   n_real, n_known)

    xk_t = known_real.transpose(1, 2, 0)[..., None]
    xo4 = observed[..., None]
    w4 = real_W[None, :, None, :]
    b4 = real_b[None, :, None, :]

    full = lambda shape: pl.BlockSpec(shape, lambda *a: (0,) * len(shape))
    OB = 32
    obs_p = pl.pallas_call(
        _obs_body,
        grid=(B // OB,),
        in_specs=[
            pl.BlockSpec((OB, T, n_obs, 1), lambda r: (r, 0, 0, 0)),
            full((n_obs, L)),
            full((n_obs, L)),
        ],
        out_specs=pl.BlockSpec((OB, T, n_obs, L), lambda r: (r, 0, 0, 0)),
        out_shape=jax.ShapeDtypeStruct((B, T, n_obs, L), jnp.float32),
    )(xo4, obs_W, obs_b)

    BB = 32
    known_p = pl.pallas_call(
        _known_dense_body,
        grid=(B // BB,),
        in_specs=[
            pl.BlockSpec((T, n_real, BB, 1), lambda r: (0, 0, r, 0)),
            full((1, n_real, 1, L)),
            full((1, n_real, 1, L)),
            pl.BlockSpec(memory_space=pl.ANY),
        ],
        out_specs=pl.BlockSpec((T, n_real, BB, L), lambda r: (0, 0, r, 0)),
        out_shape=jax.ShapeDtypeStruct((T, n_known, B, L), jnp.float32),
        input_output_aliases={3: 0},
    )(xk_t, w4, b4, catp)

    return (static_embs,
            jnp.transpose(known_p, (2, 0, 3, 1)),
            jnp.swapaxes(obs_p, 2, 3))
